# baseline (speedup 1.0000x reference)
import functools

import jax
import jax.numpy as jnp
from jax import lax
from jax.experimental import pallas as pl
from jax.experimental.pallas import tpu as pltpu
from jax.experimental.pallas import tpu_sc as plsc

NUM_DOMAINS = 26
BATCH = 16384
NC, NS, L = 1, 16, 16
NW = NC * NS
CHUNK = BATCH // NW
STEPS = CHUNK // L

_LOG1P_COEFFS = (
    0.8109301924705505, -0.1442633867263794, 0.033152297139167786,
    -0.008463365025818348, 0.0022894551511853933, -0.0006334423669613898,
    0.0001813510898500681, -6.614260200876743e-05, 2.02578266907949e-05,
)


def _softplus(w):
    u = jnp.exp(-jnp.abs(w))
    t = 2.0 * u - 1.0
    q = jnp.full_like(t, _LOG1P_COEFFS[-1])
    for c in _LOG1P_COEFFS[-2::-1]:
        q = q * t + c
    return jnp.maximum(w, 0.0) + u * q


_sc_mesh = plsc.VectorSubcoreMesh(
    core_axis_name="c", subcore_axis_name="s", num_cores=NC, num_subcores=NS
)


@functools.partial(
    pl.kernel,
    out_type=jax.ShapeDtypeStruct((BATCH,), jnp.float32),
    mesh=_sc_mesh,
    scratch_types=[
        pltpu.VMEM((CHUNK,), jnp.int32),
        pltpu.VMEM((CHUNK,), jnp.float32),
        pltpu.VMEM((CHUNK,), jnp.float32),
        pltpu.VMEM((NUM_DOMAINS,), jnp.float32),
        pltpu.VMEM((2 * L,), jnp.float32),
        pltpu.VMEM_SHARED((NUM_DOMAINS,), jnp.float32),
        pltpu.SemaphoreType.DMA,
        pltpu.SemaphoreType.DMA,
    ],
    compiler_params=pltpu.CompilerParams(needs_layout_passes=False),
)
def _sc_kernel(idx_hbm, x_hbm, raw_hbm, out_hbm,
               idx_v, x_v, out_v, raw_v, tab_v, raw_sh, sem0, sem1):
    wid = lax.axis_index("s") * NC + lax.axis_index("c")
    base = wid * CHUNK
    cp_idx = pltpu.async_copy(idx_hbm.at[pl.ds(base, CHUNK)], idx_v, sem0)
    cp_x = pltpu.async_copy(x_hbm.at[pl.ds(base, CHUNK)], x_v, sem0)

    @pl.when(wid == 0)
    def _():
        pltpu.sync_copy(raw_hbm, raw_sh)

    plsc.subcore_barrier()
    pltpu.sync_copy(raw_sh, raw_v)

    lane = lax.broadcasted_iota(jnp.int32, (L,), 0)
    idx_hi = jnp.minimum(lane + L, NUM_DOMAINS - 1)
    w_lo = plsc.load_gather(raw_v, [lane])
    w_hi = plsc.load_gather(raw_v, [idx_hi])
    mask_hi = (lane + L) < NUM_DOMAINS
    sp_lo = _softplus(w_lo)
    sp_hi = jnp.where(mask_hi, _softplus(w_hi), 0.0)
    tab_v[pl.ds(0, L)] = sp_lo
    tab_v[pl.ds(L, L)] = sp_hi
    total = jnp.broadcast_to(jnp.sum(sp_lo) + jnp.sum(sp_hi), (L,))
    scale = NUM_DOMAINS / total

    cp_idx.wait()
    cp_x.wait()
    for i in range(STEPS):
        sl = pl.ds(i * L, L)
        out_v[sl] = x_v[sl] * scale * plsc.load_gather(tab_v, [idx_v[sl]])
    cp_out = pltpu.async_copy(out_v, out_hbm.at[pl.ds(base, CHUNK)], sem1)
    cp_out.wait()


def kernel(idxes, x, raw_weights):
    out = _sc_kernel(idxes, x.reshape(BATCH), raw_weights)
    return out.reshape(BATCH, 1)


# speedup vs baseline: 1.0441x; 1.0441x over previous
import functools

import jax
import jax.numpy as jnp
from jax import lax
from jax.experimental import pallas as pl
from jax.experimental.pallas import tpu as pltpu
from jax.experimental.pallas import tpu_sc as plsc

NUM_DOMAINS = 26
BATCH = 16384
NC, NS, L = 1, 16, 16
NW = NC * NS
CHUNK = BATCH // NW
STEPS = CHUNK // L

_LOG1P_COEFFS = (
    0.8109301924705505, -0.1442633867263794, 0.033152297139167786,
    -0.008463365025818348, 0.0022894551511853933, -0.0006334423669613898,
    0.0001813510898500681, -6.614260200876743e-05, 2.02578266907949e-05,
)


def _softplus(w):
    u = jnp.exp(-jnp.abs(w))
    t = 2.0 * u - 1.0
    q = jnp.full_like(t, _LOG1P_COEFFS[-1])
    for c in _LOG1P_COEFFS[-2::-1]:
        q = q * t + c
    return jnp.maximum(w, 0.0) + u * q


_sc_mesh = plsc.VectorSubcoreMesh(
    core_axis_name="c", subcore_axis_name="s", num_cores=NC, num_subcores=NS
)


@functools.partial(
    pl.kernel,
    out_type=jax.ShapeDtypeStruct((BATCH,), jnp.float32),
    mesh=_sc_mesh,
    scratch_types=[
        pltpu.VMEM((CHUNK,), jnp.int32),
        pltpu.VMEM((CHUNK,), jnp.float32),
        pltpu.VMEM((NUM_DOMAINS,), jnp.float32),
        pltpu.VMEM((2 * L,), jnp.float32),
        pltpu.VMEM_SHARED((NUM_DOMAINS,), jnp.float32),
        pltpu.SemaphoreType.DMA,
        pltpu.SemaphoreType.DMA,
    ],
    compiler_params=pltpu.CompilerParams(needs_layout_passes=False),
)
def _sc_kernel(idx_hbm, x_hbm, raw_hbm, out_hbm,
               idx_v, x_v, raw_v, tab_v, raw_sh, sem0, sem1):
    wid = lax.axis_index("s") * NC + lax.axis_index("c")
    base = wid * CHUNK
    cp_idx = pltpu.async_copy(idx_hbm.at[pl.ds(base, CHUNK)], idx_v, sem0)
    cp_x = pltpu.async_copy(x_hbm.at[pl.ds(base, CHUNK)], x_v, sem0)

    @pl.when(wid == 0)
    def _():
        pltpu.sync_copy(raw_hbm, raw_sh)

    plsc.subcore_barrier()
    pltpu.sync_copy(raw_sh, raw_v)

    lane = lax.broadcasted_iota(jnp.int32, (L,), 0)
    idx_hi = jnp.minimum(lane + L, NUM_DOMAINS - 1)
    w_lo = plsc.load_gather(raw_v, [lane])
    w_hi = plsc.load_gather(raw_v, [idx_hi])
    mask_hi = (lane + L) < NUM_DOMAINS
    sp_lo = _softplus(w_lo)
    sp_hi = jnp.where(mask_hi, _softplus(w_hi), 0.0)
    total = jnp.broadcast_to(jnp.sum(sp_lo) + jnp.sum(sp_hi), (L,))
    scale = NUM_DOMAINS / total
    tab_v[pl.ds(0, L)] = sp_lo * scale
    tab_v[pl.ds(L, L)] = sp_hi * scale

    cp_idx.wait()
    cp_x.wait()

    @plsc.parallel_loop(0, CHUNK, step=L, unroll=8)
    def _(off):
        sl = pl.ds(off, L)
        x_v[sl] = x_v[sl] * plsc.load_gather(tab_v, [idx_v[sl]])

    cp_out = pltpu.async_copy(x_v, out_hbm.at[pl.ds(base, CHUNK)], sem1)
    cp_out.wait()


def kernel(idxes, x, raw_weights):
    out = _sc_kernel(idxes, x.reshape(BATCH), raw_weights)
    return out.reshape(BATCH, 1)
